# baseline (speedup 1.0000x reference)
import functools

import jax
import jax.numpy as jnp
import numpy as np
from jax import lax
from jax.experimental import pallas as pl
from jax.experimental.pallas import tpu as pltpu
from jax.experimental.pallas import tpu_sc as plsc

B = 1024
T = 200
D = 32
M = 1_000_000
BASE = 257
SEG = 256
HASH_VECS = 13


def _mod_m(h):
    q = (h.astype(jnp.float32) * np.float32(1.0 / M)).astype(jnp.int32)
    r = h - q * M
    r = jnp.where(r < 0, r + M, r)
    r = jnp.where(r >= M, r - M, r)
    return r


def _body(x_hbm, t2_hbm, t3_hbm, t4_hbm, out_hbm,
          x_v, idx_v, g_v, out_v, gs0, gs1, xs0, xs1, os0, os1):
    info = plsc.get_sparse_core_info()
    nc = info.num_cores
    wid = lax.axis_index("s") * nc + lax.axis_index("c")
    nw = nc * info.num_subcores
    rows_per_w = B // nw
    half = rows_per_w // 2
    b_base = wid * rows_per_w

    zi = jnp.zeros((16,), jnp.int32)

    for bb in range(2):
        x_v[bb, pl.ds(200, 16)] = zi
        x_v[bb, pl.ds(208, 16)] = zi

    def load_x(b, buf):
        return pltpu.async_copy(
            x_hbm.at[pl.ds(b * T, T)], x_v.at[buf, pl.ds(0, T)], (xs0, xs1)[buf]
        )

    def hash_row(buf):
        for v in range(HASH_VECS):
            i0 = v * 16
            x0 = x_v[buf, pl.ds(i0, 16)]
            x1 = x_v[buf, pl.ds(i0 + 1, 16)]
            x2 = x_v[buf, pl.ds(i0 + 2, 16)]
            x3 = x_v[buf, pl.ds(i0 + 3, 16)]
            h2 = _mod_m(x0 * BASE + x1)
            idx_v[buf, pl.ds(i0, 16)] = h2
            h3 = _mod_m(h2 * BASE + x2)
            idx_v[buf, pl.ds(SEG + i0, 16)] = h3
            h4 = _mod_m(h3 * BASE + x3)
            idx_v[buf, pl.ds(2 * SEG + i0, 16)] = h4

    def _gather_args(buf):
        sem = (gs0, gs1)[buf]
        for s, tbl in enumerate((t2_hbm, t3_hbm, t4_hbm)):
            for off, ln in ((0, 128), (128, 80)):
                yield (
                    tbl.at[idx_v.at[buf, pl.ds(s * SEG + off, ln)]],
                    g_v.at[buf, pl.ds(s * SEG + off, ln)],
                    sem,
                )

    def fire_gathers(buf):
        for src, dst, sem in _gather_args(buf):
            pltpu.async_copy(src, dst, sem)

    def drain_gathers(buf):
        for src, dst, sem in _gather_args(buf):
            pltpu.make_async_copy(src, dst, sem).wait()

    def add_pass(buf, b):
        out_v[buf, 0] = jnp.zeros((D,), jnp.bfloat16)
        out_v[buf, 1] = g_v[buf, 0]
        out_v[buf, 2] = g_v[buf, 1] + g_v[buf, SEG]

        def step(t, carry):
            e2 = g_v[buf, t - 1]
            e3 = g_v[buf, t + (SEG - 2)]
            e4 = g_v[buf, t + (2 * SEG - 3)]
            out_v[buf, t] = e2 + e3 + e4
            return carry

        lax.fori_loop(3, T, step, 0, unroll=4)
        return pltpu.async_copy(
            out_v.at[buf], out_hbm.at[pl.ds(b * T, T)], (os0, os1)[buf]
        )

    pltpu.sync_copy(x_hbm.at[pl.ds(b_base * T, T)], x_v.at[0, pl.ds(0, T)])
    hash_row(0)
    fire_gathers(0)
    load_x(b_base + 1, 1)

    def pipeline(k, carry):
        b0 = b_base + 2 * k
        pltpu.make_async_copy(
            x_hbm.at[pl.ds(b0 * T, T)], x_v.at[1, pl.ds(0, T)], xs1
        ).wait()
        hash_row(1)
        drain_gathers(0)
        fire_gathers(1)

        @pl.when(k < half - 1)
        def _():
            load_x(b0 + 2, 0)

        @pl.when(k > 0)
        def _():
            pltpu.make_async_copy(
                out_hbm.at[pl.ds(0, T)], out_v.at[0], os0
            ).wait()

        add_pass(0, b0)

        @pl.when(k < half - 1)
        def _():
            pltpu.make_async_copy(
                x_hbm.at[pl.ds(b0 * T, T)], x_v.at[0, pl.ds(0, T)], xs0
            ).wait()
            hash_row(0)
        drain_gathers(1)

        @pl.when(k < half - 1)
        def _():
            fire_gathers(0)
            load_x(b0 + 3, 1)

        @pl.when(k > 0)
        def _():
            pltpu.make_async_copy(
                out_hbm.at[pl.ds(0, T)], out_v.at[1], os1
            ).wait()

        add_pass(1, b0 + 1)
        return carry

    lax.fori_loop(0, half, pipeline, 0)

    pltpu.make_async_copy(out_hbm.at[pl.ds(0, T)], out_v.at[0], os0).wait()
    pltpu.make_async_copy(out_hbm.at[pl.ds(0, T)], out_v.at[1], os1).wait()


@jax.jit
def kernel(x, table_2, table_3, table_4):
    mesh = plsc.VectorSubcoreMesh(core_axis_name="c", subcore_axis_name="s")
    run = functools.partial(
        pl.kernel,
        out_type=jax.ShapeDtypeStruct((B * T, D), jnp.bfloat16),
        mesh=mesh,
        compiler_params=pltpu.CompilerParams(use_tc_tiling_on_sc=False),
        scratch_types=[
            pltpu.VMEM((2, 224), jnp.int32),
            pltpu.VMEM((2, 3 * SEG), jnp.int32),
            pltpu.VMEM((2, 3 * SEG, D), jnp.bfloat16),
            pltpu.VMEM((2, T, D), jnp.bfloat16),
            pltpu.SemaphoreType.DMA,
            pltpu.SemaphoreType.DMA,
            pltpu.SemaphoreType.DMA,
            pltpu.SemaphoreType.DMA,
            pltpu.SemaphoreType.DMA,
            pltpu.SemaphoreType.DMA,
        ],
    )(_body)
    inv3 = np.float32(1.0 / 3.0)
    out = run(
        x.reshape(-1),
        (table_2 * inv3).astype(jnp.bfloat16),
        (table_3 * inv3).astype(jnp.bfloat16),
        (table_4 * inv3).astype(jnp.bfloat16),
    )
    return out.astype(jnp.float32).reshape(B, T, D)


# speedup vs baseline: 1.6355x; 1.6355x over previous
import functools

import jax
import jax.numpy as jnp
import numpy as np
from jax import lax
from jax.experimental import pallas as pl
from jax.experimental.pallas import tpu as pltpu
from jax.experimental.pallas import tpu_sc as plsc

B = 1024
T = 200
D = 32
M = 1_000_000
BASE = 257
SEG = 256
HASH_VECS = 13
INV3 = np.float32(1.0 / 3.0)


def _mod_m(h):
    q = (h.astype(jnp.float32) * np.float32(1.0 / M)).astype(jnp.int32)
    r = h - q * M
    r = jnp.where(r < 0, r + M, r)
    r = jnp.where(r >= M, r - M, r)
    return r


def _body(x_hbm, t2_hbm, t3_hbm, t4_hbm, out_hbm,
          x_v, idx_v, g_v, out_v, gs0, gs1, xs0, xs1, os0, os1):
    info = plsc.get_sparse_core_info()
    nc = info.num_cores
    wid = lax.axis_index("s") * nc + lax.axis_index("c")
    nw = nc * info.num_subcores
    rows_per_w = B // nw
    half = rows_per_w // 2
    b_base = wid * rows_per_w

    zi = jnp.zeros((16,), jnp.int32)

    for bb in range(2):
        x_v[bb, pl.ds(200, 16)] = zi
        x_v[bb, pl.ds(208, 16)] = zi

    def load_x(b, buf):
        return pltpu.async_copy(
            x_hbm.at[pl.ds(b * T, T)], x_v.at[buf, pl.ds(0, T)], (xs0, xs1)[buf]
        )

    def hash_row(buf):
        for v in range(HASH_VECS):
            i0 = v * 16
            x0 = x_v[buf, pl.ds(i0, 16)]
            x1 = x_v[buf, pl.ds(i0 + 1, 16)]
            x2 = x_v[buf, pl.ds(i0 + 2, 16)]
            x3 = x_v[buf, pl.ds(i0 + 3, 16)]
            h2 = _mod_m(x0 * BASE + x1)
            idx_v[buf, pl.ds(i0, 16)] = h2
            h3 = _mod_m(h2 * BASE + x2)
            idx_v[buf, pl.ds(SEG + i0, 16)] = h3
            h4 = _mod_m(h3 * BASE + x3)
            idx_v[buf, pl.ds(2 * SEG + i0, 16)] = h4

    def _gather_args(buf):
        sem = (gs0, gs1)[buf]
        for s, tbl in enumerate((t2_hbm, t3_hbm, t4_hbm)):
            for off, ln in ((0, 128), (128, 80)):
                yield (
                    tbl.at[idx_v.at[buf, pl.ds(s * SEG + off, ln)]],
                    g_v.at[buf, pl.ds(s * SEG + off, ln)],
                    sem,
                )

    def fire_gathers(buf):
        for src, dst, sem in _gather_args(buf):
            pltpu.async_copy(src, dst, sem)

    def drain_gathers(buf):
        for src, dst, sem in _gather_args(buf):
            pltpu.make_async_copy(src, dst, sem).wait()

    def add_pass(buf, b):
        zf = jnp.zeros((16,), jnp.float32)
        out_v[buf, 0, pl.ds(0, 16)] = zf
        out_v[buf, 0, pl.ds(16, 16)] = zf
        for hh in (0, 16):
            out_v[buf, 1, pl.ds(hh, 16)] = g_v[buf, 0, pl.ds(hh, 16)] * INV3
            out_v[buf, 2, pl.ds(hh, 16)] = (
                g_v[buf, 1, pl.ds(hh, 16)] + g_v[buf, SEG, pl.ds(hh, 16)]
            ) * INV3

        def step(t, carry):
            for hh in (0, 16):
                e2 = g_v[buf, t - 1, pl.ds(hh, 16)]
                e3 = g_v[buf, t + (SEG - 2), pl.ds(hh, 16)]
                e4 = g_v[buf, t + (2 * SEG - 3), pl.ds(hh, 16)]
                out_v[buf, t, pl.ds(hh, 16)] = (e2 + e3 + e4) * INV3
            return carry

        lax.fori_loop(3, T, step, 0, unroll=4)
        return pltpu.async_copy(
            out_v.at[buf], out_hbm.at[pl.ds(b * T, T)], (os0, os1)[buf]
        )

    pltpu.sync_copy(x_hbm.at[pl.ds(b_base * T, T)], x_v.at[0, pl.ds(0, T)])
    hash_row(0)
    fire_gathers(0)
    load_x(b_base + 1, 1)

    def pipeline(k, carry):
        b0 = b_base + 2 * k
        pltpu.make_async_copy(
            x_hbm.at[pl.ds(b0 * T, T)], x_v.at[1, pl.ds(0, T)], xs1
        ).wait()
        hash_row(1)
        drain_gathers(0)
        fire_gathers(1)

        @pl.when(k < half - 1)
        def _():
            load_x(b0 + 2, 0)

        @pl.when(k > 0)
        def _():
            pltpu.make_async_copy(
                out_hbm.at[pl.ds(0, T)], out_v.at[0], os0
            ).wait()

        add_pass(0, b0)

        @pl.when(k < half - 1)
        def _():
            pltpu.make_async_copy(
                x_hbm.at[pl.ds(b0 * T, T)], x_v.at[0, pl.ds(0, T)], xs0
            ).wait()
            hash_row(0)
        drain_gathers(1)

        @pl.when(k < half - 1)
        def _():
            fire_gathers(0)
            load_x(b0 + 3, 1)

        @pl.when(k > 0)
        def _():
            pltpu.make_async_copy(
                out_hbm.at[pl.ds(0, T)], out_v.at[1], os1
            ).wait()

        add_pass(1, b0 + 1)
        return carry

    lax.fori_loop(0, half, pipeline, 0)

    pltpu.make_async_copy(out_hbm.at[pl.ds(0, T)], out_v.at[0], os0).wait()
    pltpu.make_async_copy(out_hbm.at[pl.ds(0, T)], out_v.at[1], os1).wait()


@jax.jit
def kernel(x, table_2, table_3, table_4):
    mesh = plsc.VectorSubcoreMesh(core_axis_name="c", subcore_axis_name="s")
    run = functools.partial(
        pl.kernel,
        out_type=jax.ShapeDtypeStruct((B * T, D), jnp.float32),
        mesh=mesh,
        compiler_params=pltpu.CompilerParams(use_tc_tiling_on_sc=False),
        scratch_types=[
            pltpu.VMEM((2, 224), jnp.int32),
            pltpu.VMEM((2, 3 * SEG), jnp.int32),
            pltpu.VMEM((2, 3 * SEG, D), jnp.float32),
            pltpu.VMEM((2, T, D), jnp.float32),
            pltpu.SemaphoreType.DMA,
            pltpu.SemaphoreType.DMA,
            pltpu.SemaphoreType.DMA,
            pltpu.SemaphoreType.DMA,
            pltpu.SemaphoreType.DMA,
            pltpu.SemaphoreType.DMA,
        ],
    )(_body)
    out = run(x.reshape(-1), table_2, table_3, table_4)
    return out.reshape(B, T, D)
